# baseline (speedup 1.0000x reference)
import functools

import jax
import jax.numpy as jnp
from jax.experimental import pallas as pl
from jax.experimental.pallas import tpu as pltpu

D_MODEL = 1024
N_CLUSTERS = 64
TOP_K = 8
BASE_TEMP = 2.0
SEQ_LEN = 2048
PRED_LEN = 512

_TEMP = BASE_TEMP * (1.0 + PRED_LEN / SEQ_LEN)
_INV_TEMP = 1.0 / _TEMP
_NCHUNK = 4


def _start_in_copies(x_hbm, xbuf, x_sem, tile, slot, tile_rows, half):
    for h in range(_NCHUNK):
        pltpu.make_async_copy(
            x_hbm.at[pl.ds(tile * tile_rows + h * half, half), :],
            xbuf.at[slot, pl.ds(h * half, half), :],
            x_sem.at[slot, h]).start()


def _fused_kernel(x_hbm, d_ref, xc_hbm, xr_hbm, aux_ref, q_buf, acc_ref,
                  xbuf, cbuf, rbuf, x_sem, c_sem, r_sem,
                  *, tile_rows, n_rows, n_tiles):
    i = pl.program_id(0)
    slot = jax.lax.rem(i, 2)
    nslot = jax.lax.rem(i + 1, 2)
    half = tile_rows // _NCHUNK
    d = d_ref[...]

    @pl.when(i == 0)
    def _():
        _start_in_copies(x_hbm, xbuf, x_sem, 0, 0, tile_rows, half)

    @pl.when(i + 1 < n_tiles)
    def _():
        _start_in_copies(x_hbm, xbuf, x_sem, i + 1, nslot, tile_rows, half)

    for h in range(_NCHUNK):
        rows = pl.ds(i * tile_rows + h * half, half)
        buf_rows = pl.ds(h * half, half)

        pltpu.make_async_copy(
            x_hbm.at[rows], xbuf.at[slot, buf_rows, :],
            x_sem.at[slot, h]).wait()
        x_h = xbuf[slot, buf_rows, :]

        scores_t = jax.lax.dot_general(
            d, x_h, (((1,), (1,)), ((), ())),
            preferred_element_type=jnp.float32)
        st = scores_t * _INV_TEMP

        m0 = jnp.max(st, axis=0, keepdims=True)
        e = jnp.exp(st - m0)
        q = e * (1.0 / jnp.sum(e, axis=0, keepdims=True))
        q_buf[:, pl.ds(i * tile_rows + h * half, half)] = q

        if h == 0:
            @pl.when(i == 0)
            def _():
                acc_ref[...] = q

            @pl.when(i > 0)
            def _():
                acc_ref[...] = acc_ref[...] + q
        else:
            acc_ref[...] = acc_ref[...] + q

        k = st.shape[0]
        iota = jax.lax.broadcasted_iota(jnp.int32, st.shape, 0)
        work = st
        neg_inf = jnp.float32(-jnp.inf)
        m = m0
        for _r in range(TOP_K):
            is_m = work == m
            idx = jnp.min(jnp.where(is_m, iota, k), axis=0, keepdims=True)
            work = jnp.where(iota == idx, neg_inf, work)
            if _r < TOP_K - 1:
                m = jnp.max(work, axis=0, keepdims=True)

        ew = jnp.where(work == neg_inf, e, 0.0)
        w = ew * (1.0 / jnp.sum(ew, axis=0, keepdims=True))

        @pl.when(i >= 2)
        def _():
            prev_rows = pl.ds((i - 2) * tile_rows + h * half, half)
            pltpu.make_async_copy(
                cbuf.at[slot, buf_rows, :], xc_hbm.at[prev_rows],
                c_sem.at[slot, h]).wait()
            pltpu.make_async_copy(
                rbuf.at[slot, buf_rows, :], xr_hbm.at[prev_rows],
                r_sem.at[slot, h]).wait()

        xc = jax.lax.dot_general(
            w, d, (((0,), (0,)), ((), ())),
            preferred_element_type=jnp.float32)
        cbuf[slot, buf_rows, :] = xc
        pltpu.make_async_copy(
            cbuf.at[slot, buf_rows, :], xc_hbm.at[rows],
            c_sem.at[slot, h]).start()

        rbuf[slot, buf_rows, :] = x_h - xc
        pltpu.make_async_copy(
            rbuf.at[slot, buf_rows, :], xr_hbm.at[rows],
            r_sem.at[slot, h]).start()

    @pl.when(i == 1)
    def _():
        gram = jax.lax.dot_general(
            d, d, (((1,), (1,)), ((), ())),
            preferred_element_type=jnp.float32)
        kk = gram.shape[0]
        r_i = jax.lax.broadcasted_iota(jnp.int32, gram.shape, 0)
        c_i = jax.lax.broadcasted_iota(jnp.int32, gram.shape, 1)
        eye = jnp.where(r_i == c_i, 1.0, 0.0).astype(gram.dtype)
        diff = gram - eye
        ortho = jnp.sum(diff * diff) / (kk * kk)
        aux_ref[0, 0] = 0.1 * ortho

    @pl.when(i == n_tiles - 1)
    def _():
        qf = q_buf[...]
        csum = jnp.sum(acc_ref[...], axis=1, keepdims=True)
        weight = (qf * qf) / csum
        rowsum = jnp.sum(weight, axis=0, keepdims=True)
        p = weight / rowsum
        kl_elem = p * (jnp.log(qf) - jnp.log(csum) - jnp.log(rowsum))
        kl = jnp.sum(kl_elem) / n_rows
        aux_ref[0, 0] = aux_ref[0, 0] + kl * (SEQ_LEN / PRED_LEN)

        for hh in range(_NCHUNK):
            prev_rows = pl.ds((i - 1) * tile_rows + hh * half, half)
            last_rows = pl.ds(i * tile_rows + hh * half, half)
            bh = pl.ds(hh * half, half)
            pltpu.make_async_copy(
                cbuf.at[nslot, bh, :], xc_hbm.at[prev_rows],
                c_sem.at[nslot, hh]).wait()
            pltpu.make_async_copy(
                rbuf.at[nslot, bh, :], xr_hbm.at[prev_rows],
                r_sem.at[nslot, hh]).wait()
            pltpu.make_async_copy(
                cbuf.at[slot, bh, :], xc_hbm.at[last_rows],
                c_sem.at[slot, hh]).wait()
            pltpu.make_async_copy(
                rbuf.at[slot, bh, :], xr_hbm.at[last_rows],
                r_sem.at[slot, hh]).wait()


def kernel(x, dictionary):
    B, N, D = x.shape
    K = dictionary.shape[0]
    n_rows = B * N
    tile_rows = 1024
    n_tiles = n_rows // tile_rows
    xf = x.reshape(n_rows, D)

    out_types = (
        jax.ShapeDtypeStruct((n_rows, D), jnp.float32),
        jax.ShapeDtypeStruct((n_rows, D), jnp.float32),
        jax.ShapeDtypeStruct((1, 1), jnp.float32),
    )
    xc, xr, aux = pl.pallas_call(
        functools.partial(_fused_kernel, tile_rows=tile_rows,
                          n_rows=n_rows, n_tiles=n_tiles),
        grid=(n_tiles,),
        in_specs=[
            pl.BlockSpec(memory_space=pl.ANY),
            pl.BlockSpec((K, D), lambda i: (0, 0)),
        ],
        out_specs=(
            pl.BlockSpec(memory_space=pl.ANY),
            pl.BlockSpec(memory_space=pl.ANY),
            pl.BlockSpec(memory_space=pltpu.SMEM),
        ),
        out_shape=out_types,
        scratch_shapes=[
            pltpu.VMEM((K, n_rows), jnp.float32),
            pltpu.VMEM((K, tile_rows // 4), jnp.float32),
            pltpu.VMEM((2, tile_rows, D), jnp.float32),
            pltpu.VMEM((2, tile_rows, D), jnp.float32),
            pltpu.VMEM((2, tile_rows, D), jnp.float32),
            pltpu.SemaphoreType.DMA((2, 4)),
            pltpu.SemaphoreType.DMA((2, 4)),
            pltpu.SemaphoreType.DMA((2, 4)),
        ],
    )(xf, dictionary)

    return (xc.reshape(B, N, D), xr.reshape(B, N, D), aux[0, 0])


# speedup vs baseline: 1.3359x; 1.3359x over previous
import jax
import jax.numpy as jnp
from jax.experimental import pallas as pl
from jax.experimental.pallas import tpu as pltpu

D_MODEL = 1024
N_CLUSTERS = 64
TOP_K = 8
BASE_TEMP = 2.0
SEQ_LEN = 2048
PRED_LEN = 512

_TEMP = BASE_TEMP * (1.0 + PRED_LEN / SEQ_LEN)
_INV_TEMP = 1.0 / _TEMP

_CHUNK = 512
_N_CHUNKS = 8
_RING = 4


def _fused_kernel(x_hbm, d_ref, xc_hbm, xr_hbm, aux_ref, q_buf, acc_ref,
                  xbuf, cbuf, rbuf, x_sem, c_sem, r_sem):
    d = d_ref[...]

    for c in range(_N_CHUNKS):
        pltpu.make_async_copy(
            x_hbm.at[pl.ds(c * _CHUNK, _CHUNK), :],
            xbuf.at[c],
            x_sem.at[c]).start()

    gram = jax.lax.dot_general(
        d, d, (((1,), (1,)), ((), ())),
        preferred_element_type=jnp.float32)
    kk = gram.shape[0]
    r_i = jax.lax.broadcasted_iota(jnp.int32, gram.shape, 0)
    c_i = jax.lax.broadcasted_iota(jnp.int32, gram.shape, 1)
    eye = jnp.where(r_i == c_i, 1.0, 0.0).astype(gram.dtype)
    diff = gram - eye
    ortho = jnp.sum(diff * diff) / (kk * kk)

    for c in range(_N_CHUNKS):
        pltpu.make_async_copy(
            x_hbm.at[pl.ds(c * _CHUNK, _CHUNK), :],
            xbuf.at[c],
            x_sem.at[c]).wait()
        x_h = xbuf[c]

        scores_t = jax.lax.dot_general(
            d, x_h, (((1,), (1,)), ((), ())),
            preferred_element_type=jnp.float32)
        st = scores_t * _INV_TEMP

        m0 = jnp.max(st, axis=0, keepdims=True)
        e = jnp.exp(st - m0)
        q = e * (1.0 / jnp.sum(e, axis=0, keepdims=True))
        q_buf[:, pl.ds(c * _CHUNK, _CHUNK)] = q
        if c == 0:
            acc_ref[...] = q
        else:
            acc_ref[...] = acc_ref[...] + q

        k = st.shape[0]
        iota = jax.lax.broadcasted_iota(jnp.int32, st.shape, 0)
        work = st
        neg_inf = jnp.float32(-jnp.inf)
        m = m0
        for _r in range(TOP_K):
            is_m = work == m
            idx = jnp.min(jnp.where(is_m, iota, k), axis=0, keepdims=True)
            work = jnp.where(iota == idx, neg_inf, work)
            if _r < TOP_K - 1:
                m = jnp.max(work, axis=0, keepdims=True)

        ew = jnp.where(work == neg_inf, e, 0.0)
        w = ew * (1.0 / jnp.sum(ew, axis=0, keepdims=True))

        slot = c % _RING
        if c >= _RING:
            prev = pl.ds((c - _RING) * _CHUNK, _CHUNK)
            pltpu.make_async_copy(
                cbuf.at[slot], xc_hbm.at[prev], c_sem.at[slot]).wait()
            pltpu.make_async_copy(
                rbuf.at[slot], xr_hbm.at[prev], r_sem.at[slot]).wait()

        rows = pl.ds(c * _CHUNK, _CHUNK)
        xc = jax.lax.dot_general(
            w, d, (((0,), (0,)), ((), ())),
            preferred_element_type=jnp.float32)
        cbuf[slot] = xc
        pltpu.make_async_copy(
            cbuf.at[slot], xc_hbm.at[rows], c_sem.at[slot]).start()

        rbuf[slot] = x_h - xc
        pltpu.make_async_copy(
            rbuf.at[slot], xr_hbm.at[rows], r_sem.at[slot]).start()

    qf = q_buf[...]
    csum = jnp.sum(acc_ref[...], axis=1, keepdims=True)
    weight = (qf * qf) / csum
    rowsum = jnp.sum(weight, axis=0, keepdims=True)
    p = weight / rowsum
    kl_elem = p * (jnp.log(qf) - jnp.log(csum) - jnp.log(rowsum))
    kl = jnp.sum(kl_elem) / qf.shape[1]
    aux_ref[0, 0] = kl * (SEQ_LEN / PRED_LEN) + 0.1 * ortho

    for c in range(_N_CHUNKS - _RING, _N_CHUNKS):
        slot = c % _RING
        rows = pl.ds(c * _CHUNK, _CHUNK)
        pltpu.make_async_copy(
            cbuf.at[slot], xc_hbm.at[rows], c_sem.at[slot]).wait()
        pltpu.make_async_copy(
            rbuf.at[slot], xr_hbm.at[rows], r_sem.at[slot]).wait()


def kernel(x, dictionary):
    B, N, D = x.shape
    K = dictionary.shape[0]
    n_rows = B * N
    xf = x.reshape(n_rows, D)

    out_types = (
        jax.ShapeDtypeStruct((n_rows, D), jnp.float32),
        jax.ShapeDtypeStruct((n_rows, D), jnp.float32),
        jax.ShapeDtypeStruct((1, 1), jnp.float32),
    )
    xc, xr, aux = pl.pallas_call(
        _fused_kernel,
        in_specs=[
            pl.BlockSpec(memory_space=pl.ANY),
            pl.BlockSpec(memory_space=pltpu.VMEM),
        ],
        out_specs=(
            pl.BlockSpec(memory_space=pl.ANY),
            pl.BlockSpec(memory_space=pl.ANY),
            pl.BlockSpec(memory_space=pltpu.SMEM),
        ),
        out_shape=out_types,
        scratch_shapes=[
            pltpu.VMEM((K, n_rows), jnp.float32),
            pltpu.VMEM((K, _CHUNK), jnp.float32),
            pltpu.VMEM((_N_CHUNKS, _CHUNK, D), jnp.float32),
            pltpu.VMEM((_RING, _CHUNK, D), jnp.float32),
            pltpu.VMEM((_RING, _CHUNK, D), jnp.float32),
            pltpu.SemaphoreType.DMA((_N_CHUNKS,)),
            pltpu.SemaphoreType.DMA((_RING,)),
            pltpu.SemaphoreType.DMA((_RING,)),
        ],
    )(xf, dictionary)

    return (xc.reshape(B, N, D), xr.reshape(B, N, D), aux[0, 0])
